# baseline (device time: 153877 ns/iter reference)
import os

import numpy as np

import jax
import jax.numpy as jnp
from jax import lax
from jax.experimental import pallas as pl
from jax.experimental.pallas import tpu as pltpu

_KV = os.environ.get("KV", "full")
DO_COMM = _KV in ("full", "comm")
DO_COMPUTE = _KV in ("full", "compute")

N_DEV = 16
N_PLANES = 4
PER_PLANE = 4
BM = 512
CHUNK = 2048
SUB = int(os.environ.get("SUBMSG", "1"))
SUBR = BM // SUB

_ORDER_TABLE = np.zeros((N_PLANES, N_DEV), np.int32)
for _p in range(N_PLANES):
    _ORDER_TABLE[_p] = sorted(
        range(N_DEV), key=lambda j: (abs(j // PER_PLANE - _p), j)
    )


def kernel(x, w_mat):
    m, k_per = x.shape
    k_tot, n = w_mat.shape
    n_chunks = m // CHUNK

    def body(order_ref, x_hbm, w_ref, out_ref, xf_ref, xb_ref, comm_ref,
             xsems, send_sems, recv_sems):
        k = pl.program_id(0)
        me = lax.axis_index("i")
        my_plane = me // PER_PLANE
        my_col = me % PER_PLANE
        j = order_ref[k]

        def send_sub(d, s):
            rdma = pltpu.make_async_remote_copy(
                src_ref=xb_ref.at[pl.ds(d * BM + s * SUBR, SUBR), :],
                dst_ref=comm_ref.at[me, pl.ds(s * SUBR, SUBR), :],
                send_sem=send_sems.at[s, d],
                recv_sem=recv_sems.at[s, me],
                device_id=(d,),
                device_id_type=pl.DeviceIdType.MESH,
            )
            rdma.start()

        @pl.when(k == 0)
        def _first():
            out_ref[...] = jnp.zeros_like(out_ref)
            pltpu.make_async_copy(
                x_hbm.at[pl.ds(0, CHUNK), :], xf_ref.at[0], xsems.at[0]
            ).start()
            for c in range(n_chunks):
                if c + 1 < n_chunks:
                    pltpu.make_async_copy(
                        x_hbm.at[pl.ds((c + 1) * CHUNK, CHUNK), :],
                        xf_ref.at[(c + 1) % 2],
                        xsems.at[(c + 1) % 2],
                    ).start()
                pltpu.make_async_copy(
                    x_hbm.at[pl.ds(c * CHUNK, CHUNK), :],
                    xf_ref.at[c % 2],
                    xsems.at[c % 2],
                ).wait()
                xb_ref[pl.ds(c * CHUNK, CHUNK), :] = (
                    xf_ref[c % 2].astype(jnp.bfloat16)
                )
            if _KV == "p2p":
                right = lax.rem(me + 1, N_DEV)
                for t in range(N_DEV - 1):
                    pltpu.make_async_remote_copy(
                        src_ref=xb_ref.at[pl.ds(t * BM, BM), :],
                        dst_ref=comm_ref.at[t],
                        send_sem=send_sems.at[0, t],
                        recv_sem=recv_sems.at[0, t],
                        device_id=(right,),
                        device_id_type=pl.DeviceIdType.MESH,
                    ).start()
                for t in range(N_DEV - 1):
                    pltpu.make_async_remote_copy(
                        src_ref=xb_ref.at[pl.ds(t * BM, BM), :],
                        dst_ref=comm_ref.at[t],
                        send_sem=send_sems.at[0, t],
                        recv_sem=recv_sems.at[0, t],
                        device_id=(me,),
                        device_id_type=pl.DeviceIdType.MESH,
                    ).wait_recv()
                for t in range(N_DEV - 1):
                    pltpu.make_async_remote_copy(
                        src_ref=xb_ref.at[pl.ds(t * BM, BM), :],
                        dst_ref=comm_ref.at[t],
                        send_sem=send_sems.at[0, t],
                        recv_sem=recv_sems.at[0, t],
                        device_id=(right,),
                        device_id_type=pl.DeviceIdType.MESH,
                    ).wait_send()
                return
            if not DO_COMM:
                return
            for t in range(1, N_DEV):
                d = lax.rem(me + t, N_DEV)
                for s in range(SUB):
                    send_sub(d, s)

        @pl.when(jnp.logical_and(j != me, DO_COMM))
        def _wait():
            for s in range(SUB):
                recv = pltpu.make_async_remote_copy(
                    src_ref=comm_ref.at[j, pl.ds(s * SUBR, SUBR), :],
                    dst_ref=comm_ref.at[j, pl.ds(s * SUBR, SUBR), :],
                    send_sem=send_sems.at[s, j],
                    recv_sem=recv_sems.at[s, j],
                    device_id=(me,),
                    device_id_type=pl.DeviceIdType.MESH,
                )
                recv.wait_recv()

        if DO_COMPUTE:
            a_own = xb_ref[pl.ds(me * BM, BM), :]
            a = jnp.where(j == me, a_own, comm_ref[j])
            wb = w_ref[...].astype(jnp.bfloat16)
            out_ref[...] += jnp.dot(a, wb, preferred_element_type=jnp.float32)

        @pl.when(jnp.logical_and(k == N_DEV - 1, DO_COMM))
        def _fin():
            for t in range(1, N_DEV):
                d = lax.rem(me + t, N_DEV)
                for s in range(SUB):
                    snd = pltpu.make_async_remote_copy(
                        src_ref=xb_ref.at[pl.ds(d * BM + s * SUBR, SUBR), :],
                        dst_ref=comm_ref.at[me, pl.ds(s * SUBR, SUBR), :],
                        send_sem=send_sems.at[s, d],
                        recv_sem=recv_sems.at[s, me],
                        device_id=(d,),
                        device_id_type=pl.DeviceIdType.MESH,
                    )
                    snd.wait_send()

        if DO_COMPUTE:
            @pl.when(k == N_DEV - 1)
            def _silu():
                y = out_ref[...]
                out_ref[...] = y * (1.0 / (1.0 + jnp.exp(-y)))

    me_out = lax.axis_index("i")
    order = jnp.mod(
        me_out - jnp.arange(N_DEV, dtype=jnp.int32), N_DEV
    ).astype(jnp.int32)

    grid_spec = pltpu.PrefetchScalarGridSpec(
        num_scalar_prefetch=1,
        grid=(N_DEV,),
        in_specs=[
            pl.BlockSpec(memory_space=pl.ANY),
            pl.BlockSpec((BM, n), lambda k, order: (order[k], 0)),
        ],
        out_specs=pl.BlockSpec((BM, n), lambda k, order: (0, 0)),
        scratch_shapes=[
            pltpu.VMEM((2, CHUNK, k_per), jnp.float32),
            pltpu.VMEM((m, k_per), jnp.bfloat16),
            pltpu.VMEM((N_DEV, BM, k_per), jnp.bfloat16),
            pltpu.SemaphoreType.DMA((2,)),
            pltpu.SemaphoreType.DMA((SUB, N_DEV)),
            pltpu.SemaphoreType.DMA((SUB, N_DEV)),
        ],
    )

    return pl.pallas_call(
        body,
        grid_spec=grid_spec,
        out_shape=jax.ShapeDtypeStruct((BM, n), jnp.float32),
        compiler_params=pltpu.CompilerParams(
            dimension_semantics=("arbitrary",),
            vmem_limit_bytes=56 * 1024 * 1024,
        ),
    )(order, x, w_mat)


# device time: 127449 ns/iter; 1.2074x vs baseline; 1.2074x over previous
import os

import numpy as np

import jax
import jax.numpy as jnp
from jax import lax
from jax.experimental import pallas as pl
from jax.experimental.pallas import tpu as pltpu

_KV = os.environ.get("KV", "full")
DO_COMM = _KV in ("full", "comm")
DO_COMPUTE = _KV in ("full", "compute")

N_DEV = 16
N_PLANES = 4
PER_PLANE = 4
BM = 512
CHUNK = 2048
SUB = int(os.environ.get("SUBMSG", "1"))
SUBR = BM // SUB

_ORDER_TABLE = np.zeros((N_PLANES, N_DEV), np.int32)
for _p in range(N_PLANES):
    _ORDER_TABLE[_p] = sorted(
        range(N_DEV), key=lambda j: (abs(j // PER_PLANE - _p), j)
    )


def kernel(x, w_mat):
    m, k_per = x.shape
    k_tot, n = w_mat.shape
    n_chunks = m // CHUNK

    def body(order_ref, x_hbm, w_ref, out_ref, xf_ref, xb_ref, comm_ref,
             xsems, send_sems, recv_sems):
        k = pl.program_id(0)
        me = lax.axis_index("i")
        my_plane = me // PER_PLANE
        my_col = me % PER_PLANE
        j = order_ref[k]

        def send_sub(d, s):
            rdma = pltpu.make_async_remote_copy(
                src_ref=xb_ref.at[pl.ds(d * BM + s * SUBR, SUBR), :],
                dst_ref=comm_ref.at[me, pl.ds(s * SUBR, SUBR), :],
                send_sem=send_sems.at[s, d],
                recv_sem=recv_sems.at[s, me],
                device_id=(d,),
                device_id_type=pl.DeviceIdType.MESH,
            )
            rdma.start()

        @pl.when(k == 0)
        def _first():
            out_ref[...] = jnp.zeros_like(out_ref)
            def chunk_idx(cs):
                return lax.rem(my_plane + 1 + cs, N_PLANES)

            pltpu.make_async_copy(
                x_hbm.at[pl.ds(chunk_idx(0) * CHUNK, CHUNK), :],
                xf_ref.at[0],
                xsems.at[0],
            ).start()
            for cs in range(n_chunks):
                cc = chunk_idx(cs)
                if cs + 1 < n_chunks:
                    pltpu.make_async_copy(
                        x_hbm.at[pl.ds(chunk_idx(cs + 1) * CHUNK, CHUNK), :],
                        xf_ref.at[(cs + 1) % 2],
                        xsems.at[(cs + 1) % 2],
                    ).start()
                pltpu.make_async_copy(
                    x_hbm.at[pl.ds(cc * CHUNK, CHUNK), :],
                    xf_ref.at[cs % 2],
                    xsems.at[cs % 2],
                ).wait()
                xb_ref[pl.ds(cc * CHUNK, CHUNK), :] = (
                    xf_ref[cs % 2].astype(jnp.bfloat16)
                )
                if DO_COMM:
                    for r in range(PER_PLANE):
                        d = cc * PER_PLANE + lax.rem(my_col + r, PER_PLANE)

                        @pl.when(d != me)
                        def _s(d=d):
                            for s in range(SUB):
                                send_sub(d, s)
            if _KV == "p2p":
                right = lax.rem(me + 1, N_DEV)
                for t in range(N_DEV - 1):
                    pltpu.make_async_remote_copy(
                        src_ref=xb_ref.at[pl.ds(t * BM, BM), :],
                        dst_ref=comm_ref.at[t],
                        send_sem=send_sems.at[0, t],
                        recv_sem=recv_sems.at[0, t],
                        device_id=(right,),
                        device_id_type=pl.DeviceIdType.MESH,
                    ).start()
                for t in range(N_DEV - 1):
                    pltpu.make_async_remote_copy(
                        src_ref=xb_ref.at[pl.ds(t * BM, BM), :],
                        dst_ref=comm_ref.at[t],
                        send_sem=send_sems.at[0, t],
                        recv_sem=recv_sems.at[0, t],
                        device_id=(me,),
                        device_id_type=pl.DeviceIdType.MESH,
                    ).wait_recv()
                for t in range(N_DEV - 1):
                    pltpu.make_async_remote_copy(
                        src_ref=xb_ref.at[pl.ds(t * BM, BM), :],
                        dst_ref=comm_ref.at[t],
                        send_sem=send_sems.at[0, t],
                        recv_sem=recv_sems.at[0, t],
                        device_id=(right,),
                        device_id_type=pl.DeviceIdType.MESH,
                    ).wait_send()
                return
        @pl.when(jnp.logical_and(j != me, DO_COMM))
        def _wait():
            for s in range(SUB):
                recv = pltpu.make_async_remote_copy(
                    src_ref=comm_ref.at[j, pl.ds(s * SUBR, SUBR), :],
                    dst_ref=comm_ref.at[j, pl.ds(s * SUBR, SUBR), :],
                    send_sem=send_sems.at[s, j],
                    recv_sem=recv_sems.at[s, j],
                    device_id=(me,),
                    device_id_type=pl.DeviceIdType.MESH,
                )
                recv.wait_recv()

        if DO_COMPUTE:
            a_own = xb_ref[pl.ds(me * BM, BM), :]
            a = jnp.where(j == me, a_own, comm_ref[j])
            wb = w_ref[...].astype(jnp.bfloat16)
            out_ref[...] += jnp.dot(a, wb, preferred_element_type=jnp.float32)

        @pl.when(jnp.logical_and(k == N_DEV - 1, DO_COMM))
        def _fin():
            for t in range(1, N_DEV):
                d = lax.rem(me + t, N_DEV)
                for s in range(SUB):
                    snd = pltpu.make_async_remote_copy(
                        src_ref=xb_ref.at[pl.ds(d * BM + s * SUBR, SUBR), :],
                        dst_ref=comm_ref.at[me, pl.ds(s * SUBR, SUBR), :],
                        send_sem=send_sems.at[s, d],
                        recv_sem=recv_sems.at[s, me],
                        device_id=(d,),
                        device_id_type=pl.DeviceIdType.MESH,
                    )
                    snd.wait_send()

        if DO_COMPUTE:
            @pl.when(k == N_DEV - 1)
            def _silu():
                y = out_ref[...]
                out_ref[...] = y * (1.0 / (1.0 + jnp.exp(-y)))

    me_out = lax.axis_index("i")
    ks = jnp.arange(N_DEV, dtype=jnp.int32)
    plane_j = jnp.mod(me_out // PER_PLANE - 1 - ks // PER_PLANE, N_PLANES)
    col_j = jnp.mod(me_out % PER_PLANE - ks % PER_PLANE, PER_PLANE)
    order = (plane_j * PER_PLANE + col_j).astype(jnp.int32)

    grid_spec = pltpu.PrefetchScalarGridSpec(
        num_scalar_prefetch=1,
        grid=(N_DEV,),
        in_specs=[
            pl.BlockSpec(memory_space=pl.ANY),
            pl.BlockSpec((BM, n), lambda k, order: (order[k], 0)),
        ],
        out_specs=pl.BlockSpec((BM, n), lambda k, order: (0, 0)),
        scratch_shapes=[
            pltpu.VMEM((2, CHUNK, k_per), jnp.float32),
            pltpu.VMEM((m, k_per), jnp.bfloat16),
            pltpu.VMEM((N_DEV, BM, k_per), jnp.bfloat16),
            pltpu.SemaphoreType.DMA((2,)),
            pltpu.SemaphoreType.DMA((SUB, N_DEV)),
            pltpu.SemaphoreType.DMA((SUB, N_DEV)),
        ],
    )

    return pl.pallas_call(
        body,
        grid_spec=grid_spec,
        out_shape=jax.ShapeDtypeStruct((BM, n), jnp.float32),
        compiler_params=pltpu.CompilerParams(
            dimension_semantics=("arbitrary",),
            vmem_limit_bytes=56 * 1024 * 1024,
        ),
    )(order, x, w_mat)
